# baseline (speedup 1.0000x reference)
import functools

import jax
import jax.numpy as jnp
from jax import lax
from jax.experimental import pallas as pl
from jax.experimental.pallas import tpu as pltpu
from jax.experimental.pallas import tpu_sc as plsc

Q = 1024
K = 131072
D = 128
TOP_K = 3
BK = 2048
PAD = 8

_VMASK = -1024
_SMASK = 1023


def _topk_body(tc_ref, wc_ref, bc_ref, mk_ref, ck_ref,
               m1_ref, m2_ref, m3_ref, ckb_ref):
    k = pl.program_id(0)

    @pl.when(k == 0)
    def _init():
        ck = lax.dot_general(tc_ref[...], wc_ref[...], (((1,), (1,)), ((), ())),
                             preferred_element_type=jnp.float32)
        ck = ck + bc_ref[...]
        ck_ref[...] = ck
        ckb_ref[...] = ck.astype(jnp.bfloat16)
        m1_ref[...] = jnp.full((Q, 128), -jnp.inf, jnp.float32)
        m2_ref[...] = jnp.full((Q, 128), -jnp.inf, jnp.float32)
        m3_ref[...] = jnp.full((Q, 128), -jnp.inf, jnp.float32)

    mk = mk_ref[...].astype(jnp.bfloat16)
    s = lax.dot_general(ckb_ref[...], mk, (((1,), (1,)), ((), ())),
                        preferred_element_type=jnp.float32)

    def key_chunk(c):
        xb = lax.bitcast_convert_type(s[:, c * 128:(c + 1) * 128], jnp.int32)
        ki = (xb & jnp.int32(_VMASK)) | (k * (BK // 128) + c)
        return lax.bitcast_convert_type(ki, jnp.float32)

    m1, m2, m3 = m1_ref[...], m2_ref[...], m3_ref[...]
    for c in range(0, BK // 128, 2):
        x1 = key_chunk(c)
        x2 = key_chunk(c + 1)
        hi = jnp.maximum(x1, x2)
        lo = jnp.minimum(x1, x2)
        u = jnp.minimum(m1, hi)
        m1 = jnp.maximum(m1, hi)
        q = jnp.maximum(m2, lo)
        v = jnp.minimum(u, q)
        m2 = jnp.maximum(u, q)
        m3 = jnp.maximum(v, m3)
    m1_ref[...] = m1
    m2_ref[...] = m2
    m3_ref[...] = m3


def _topk_call(token_content, W_content, b_content_row, mem_keys):
    return pl.pallas_call(
        _topk_body,
        grid=(K // BK,),
        in_specs=[
            pl.BlockSpec((Q, D), lambda k: (0, 0)),
            pl.BlockSpec((D, D), lambda k: (0, 0)),
            pl.BlockSpec((1, D), lambda k: (0, 0)),
            pl.BlockSpec((BK, D), lambda k: (k, 0)),
        ],
        out_specs=[
            pl.BlockSpec((Q, D), lambda k: (0, 0)),
            pl.BlockSpec((Q, 128), lambda k: (0, 0)),
            pl.BlockSpec((Q, 128), lambda k: (0, 0)),
            pl.BlockSpec((Q, 128), lambda k: (0, 0)),
        ],
        out_shape=[
            jax.ShapeDtypeStruct((Q, D), jnp.float32),
            jax.ShapeDtypeStruct((Q, 128), jnp.float32),
            jax.ShapeDtypeStruct((Q, 128), jnp.float32),
            jax.ShapeDtypeStruct((Q, 128), jnp.float32),
        ],
        scratch_shapes=[
            pltpu.VMEM((Q, D), jnp.bfloat16),
        ],
        compiler_params=pltpu.CompilerParams(
            dimension_semantics=("arbitrary",)),
    )(token_content, W_content, b_content_row, mem_keys)


def _extract_body(m1_ref, m2_ref, m3_ref, idx_ref):
    iota = lax.broadcasted_iota(jnp.int32, (Q, 128), 1)
    cur, nxt, nxt2 = m1_ref[...], m2_ref[...], m3_ref[...]
    out = []
    for t in range(TOP_K):
        mx = jnp.max(cur, axis=1, keepdims=True)
        mi = lax.bitcast_convert_type(mx, jnp.int32)
        hit = cur == mx
        lane = jnp.min(jnp.where(hit, iota, jnp.int32(128)),
                       axis=1, keepdims=True)
        out.append(((mi & jnp.int32(_SMASK)) << 7) | lane)
        if t < TOP_K - 1:
            first = hit & (iota == lane)
            cur = jnp.where(first, nxt, cur)
            nxt = jnp.where(first, nxt2, nxt)
            nxt2 = jnp.where(first, jnp.float32(-jnp.inf), nxt2)
    pad_i = jnp.zeros((Q, PAD - TOP_K), jnp.int32)
    idx_ref[...] = jnp.concatenate(out + [pad_i], axis=1)


def _extract_call(m1, m2, m3):
    return pl.pallas_call(
        _extract_body,
        out_shape=jax.ShapeDtypeStruct((Q, PAD), jnp.int32),
    )(m1, m2, m3)


def _gather_call(flat_idx, values, keys):
    B = flat_idx.shape[0]
    info = plsc.get_sparse_core_info()
    nc, ns = info.num_cores, info.num_subcores
    nw = nc * ns
    b_per_w = B // nw
    mesh = plsc.VectorSubcoreMesh(core_axis_name="c", subcore_axis_name="s")

    @functools.partial(
        pl.kernel, mesh=mesh,
        out_type=[
            jax.ShapeDtypeStruct((B, D), jnp.float32),
            jax.ShapeDtypeStruct((B, D), jnp.float32),
        ],
        scratch_types=[
            pltpu.VMEM((b_per_w,), jnp.int32),
            pltpu.VMEM((b_per_w, D), jnp.float32),
            pltpu.VMEM((b_per_w, D), jnp.float32),
            pltpu.SemaphoreType.DMA,
            pltpu.SemaphoreType.DMA,
        ],
    )
    def gather_k(idx_hbm, val_hbm, key_hbm, outv_hbm, outk_hbm,
                 idx_v, rows_v, rows_k, semv, semk):
        wid = lax.axis_index("s") * nc + lax.axis_index("c")
        base = wid * b_per_w
        pltpu.sync_copy(idx_hbm.at[pl.ds(base, b_per_w)], idx_v)
        cpv = pltpu.async_copy(val_hbm.at[idx_v], rows_v, semv)
        cpk = pltpu.async_copy(key_hbm.at[idx_v], rows_k, semk)
        cpv.wait()
        cpk.wait()
        pltpu.sync_copy(rows_v, outv_hbm.at[pl.ds(base, b_per_w)])
        pltpu.sync_copy(rows_k, outk_hbm.at[pl.ds(base, b_per_w)])

    return gather_k(flat_idx, values, keys)


def _epilogue_body(posf_ref, wpos_ref, bpos_ref, ck_ref,
                   gv0_ref, gv1_ref, gv2_ref, gk0_ref, gk1_ref, gk2_ref,
                   ts_ref, sw_ref, wg_ref, bg_ref, we_ref, be_ref, out_ref):
    ck = ck_ref[...]
    sims = [jnp.sum(ck * gk_ref[...], axis=1, keepdims=True)
            for gk_ref in (gk0_ref, gk1_ref, gk2_ref)]
    m = jnp.maximum(jnp.maximum(sims[0], sims[1]), sims[2])
    e = [jnp.exp(sv - m) for sv in sims]
    denom = e[0] + e[1] + e[2]
    sim = e[0] * gv0_ref[...] + e[1] * gv1_ref[...] + e[2] * gv2_ref[...]
    sim = sim / denom
    base = posf_ref[...] * wpos_ref[...] + bpos_ref[...]
    fe = base + sw_ref[...] * sim
    gate_in = jnp.concatenate([fe, ts_ref[...]], axis=1)
    z = lax.dot_general(gate_in, wg_ref[...], (((1,), (1,)), ((), ())),
                        preferred_element_type=jnp.float32) + bg_ref[...]
    ti = jax.nn.sigmoid(z)
    ev = lax.dot_general(fe, we_ref[...], (((1,), (1,)), ((), ())),
                         preferred_element_type=jnp.float32) + be_ref[...]
    out_ref[...] = fe + ti * ev


def _epilogue_call(pos_f, wpos_row, bpos_row, ck, gvs, gks,
                   temporal_state, sw, W_gate, bg_row, W_evol, be_row):
    return pl.pallas_call(
        _epilogue_body,
        out_shape=jax.ShapeDtypeStruct((Q, D), jnp.float32),
    )(pos_f, wpos_row, bpos_row, ck, *gvs, *gks, temporal_state,
      sw, W_gate, bg_row, W_evol, be_row)


def kernel(positions, token_content, temporal_state, mem_keys, mem_values,
           W_pos, b_pos, W_content, b_content, similarity_weight,
           W_gate, b_gate, W_evol, b_evol):
    ck, m1, m2, m3 = _topk_call(
        token_content, W_content, b_content.reshape(1, D), mem_keys)
    top_idx = _extract_call(m1, m2, m3)

    flat_idx = top_idx[:, :TOP_K].T.reshape(-1)
    gath_v, gath_k = _gather_call(flat_idx, mem_values, mem_keys)
    gvs = [gath_v[t * Q:(t + 1) * Q] for t in range(TOP_K)]
    gks = [gath_k[t * Q:(t + 1) * Q] for t in range(TOP_K)]

    pos_f = positions.astype(jnp.float32).reshape(Q, 1)
    return _epilogue_call(
        pos_f,
        W_pos.reshape(1, D),
        b_pos.reshape(1, D),
        ck,
        gvs,
        gks,
        temporal_state,
        similarity_weight.reshape(1, 1),
        W_gate,
        b_gate.reshape(1, D),
        W_evol,
        b_evol.reshape(1, D),
    )


# speedup vs baseline: 1.3206x; 1.3206x over previous
import functools

import jax
import jax.numpy as jnp
from jax import lax
from jax.experimental import pallas as pl
from jax.experimental.pallas import tpu as pltpu
from jax.experimental.pallas import tpu_sc as plsc

Q = 1024
K = 131072
D = 128
TOP_K = 3
BK = 2048
PAD = 8

N_CAND = 2 * TOP_K
_VMASK = -512
_SMASK = 511


def _topk_body(tc_ref, wc_ref, bc_ref, mk_ref, idx_ref, ck_ref, ckb_ref,
               m1_ref, m2_ref, m3_ref):
    k = pl.program_id(0)

    @pl.when(k == 0)
    def _init():
        ck = lax.dot_general(tc_ref[...], wc_ref[...], (((1,), (1,)), ((), ())),
                             preferred_element_type=jnp.float32)
        ck = ck + bc_ref[...]
        ck_ref[...] = ck
        ckb_ref[...] = ck.astype(jnp.bfloat16)
        m1_ref[...] = jnp.full((Q, 128), -jnp.inf, jnp.float32)
        m2_ref[...] = jnp.full((Q, 128), -jnp.inf, jnp.float32)
        m3_ref[...] = jnp.full((Q, 128), -jnp.inf, jnp.float32)

    mk = mk_ref[...].astype(jnp.bfloat16)
    s = lax.dot_general(ckb_ref[...], mk, (((1,), (1,)), ((), ())),
                        preferred_element_type=jnp.float32)

    def pair_key(j):
        p = jnp.maximum(s[:, (2 * j) * 128:(2 * j + 1) * 128],
                        s[:, (2 * j + 1) * 128:(2 * j + 2) * 128])
        xb = lax.bitcast_convert_type(p, jnp.int32)
        ki = (xb & jnp.int32(_VMASK)) | (k * (BK // 256) + j)
        return lax.bitcast_convert_type(ki, jnp.float32)

    m1, m2, m3 = m1_ref[...], m2_ref[...], m3_ref[...]
    for j in range(0, BK // 256, 2):
        x1 = pair_key(j)
        x2 = pair_key(j + 1)
        hi = jnp.maximum(x1, x2)
        lo = jnp.minimum(x1, x2)
        u = jnp.minimum(m1, hi)
        m1 = jnp.maximum(m1, hi)
        q = jnp.maximum(m2, lo)
        v = jnp.minimum(u, q)
        m2 = jnp.maximum(u, q)
        m3 = jnp.maximum(v, m3)
    m1_ref[...] = m1
    m2_ref[...] = m2
    m3_ref[...] = m3

    @pl.when(k == pl.num_programs(0) - 1)
    def _extract():
        iota = lax.broadcasted_iota(jnp.int32, (Q, 128), 1)
        cur, nxt, nxt2 = m1, m2, m3
        out = []
        for t in range(TOP_K):
            mx = jnp.max(cur, axis=1, keepdims=True)
            mi = lax.bitcast_convert_type(mx, jnp.int32)
            hit = cur == mx
            lane = jnp.min(jnp.where(hit, iota, jnp.int32(128)),
                           axis=1, keepdims=True)
            r0 = ((mi & jnp.int32(_SMASK)) << 8) | lane
            out.append(r0)
            out.append(r0 + jnp.int32(128))
            if t < TOP_K - 1:
                first = hit & (iota == lane)
                cur = jnp.where(first, nxt, cur)
                nxt = jnp.where(first, nxt2, nxt)
                nxt2 = jnp.where(first, jnp.float32(-jnp.inf), nxt2)
        pad_i = jnp.zeros((Q, PAD - N_CAND), jnp.int32)
        idx_ref[...] = jnp.concatenate(out + [pad_i], axis=1)


def _topk_call(token_content, W_content, b_content_row, mem_keys):
    return pl.pallas_call(
        _topk_body,
        grid=(K // BK,),
        in_specs=[
            pl.BlockSpec((Q, D), lambda k: (0, 0)),
            pl.BlockSpec((D, D), lambda k: (0, 0)),
            pl.BlockSpec((1, D), lambda k: (0, 0)),
            pl.BlockSpec((BK, D), lambda k: (k, 0)),
        ],
        out_specs=[
            pl.BlockSpec((Q, PAD), lambda k: (0, 0)),
            pl.BlockSpec((Q, D), lambda k: (0, 0)),
        ],
        out_shape=[
            jax.ShapeDtypeStruct((Q, PAD), jnp.int32),
            jax.ShapeDtypeStruct((Q, D), jnp.float32),
        ],
        scratch_shapes=[
            pltpu.VMEM((Q, D), jnp.bfloat16),
            pltpu.VMEM((Q, 128), jnp.float32),
            pltpu.VMEM((Q, 128), jnp.float32),
            pltpu.VMEM((Q, 128), jnp.float32),
        ],
        compiler_params=pltpu.CompilerParams(
            dimension_semantics=("arbitrary",)),
    )(token_content, W_content, b_content_row, mem_keys)


def _gather_call(flat_idx, values, keys):
    B = flat_idx.shape[0]
    info = plsc.get_sparse_core_info()
    nc, ns = info.num_cores, info.num_subcores
    nw = nc * ns
    b_per_w = B // nw
    mesh = plsc.VectorSubcoreMesh(core_axis_name="c", subcore_axis_name="s")

    @functools.partial(
        pl.kernel, mesh=mesh,
        out_type=[
            jax.ShapeDtypeStruct((B, D), jnp.float32),
            jax.ShapeDtypeStruct((B, D), jnp.float32),
        ],
        scratch_types=[
            pltpu.VMEM((b_per_w,), jnp.int32),
            pltpu.VMEM((b_per_w, D), jnp.float32),
            pltpu.VMEM((b_per_w, D), jnp.float32),
            pltpu.SemaphoreType.DMA,
            pltpu.SemaphoreType.DMA,
        ],
    )
    def gather_k(idx_hbm, val_hbm, key_hbm, outv_hbm, outk_hbm,
                 idx_v, rows_v, rows_k, semv, semk):
        wid = lax.axis_index("s") * nc + lax.axis_index("c")
        base = wid * b_per_w
        pltpu.sync_copy(idx_hbm.at[pl.ds(base, b_per_w)], idx_v)
        cpv = pltpu.async_copy(val_hbm.at[idx_v], rows_v, semv)
        cpk = pltpu.async_copy(key_hbm.at[idx_v], rows_k, semk)
        cpv.wait()
        cpk.wait()
        pltpu.sync_copy(rows_v, outv_hbm.at[pl.ds(base, b_per_w)])
        pltpu.sync_copy(rows_k, outk_hbm.at[pl.ds(base, b_per_w)])

    return gather_k(flat_idx, values, keys)


def _epilogue_body(posf_ref, wpos_ref, bpos_ref, ck_ref,
                   gv0_ref, gv1_ref, gv2_ref, gv3_ref, gv4_ref, gv5_ref,
                   gk0_ref, gk1_ref, gk2_ref, gk3_ref, gk4_ref, gk5_ref,
                   ts_ref, sw_ref, wg_ref, bg_ref, we_ref, be_ref, out_ref):
    ck = ck_ref[...]
    gv_refs = (gv0_ref, gv1_ref, gv2_ref, gv3_ref, gv4_ref, gv5_ref)
    sims = [jnp.sum(ck * gk_ref[...], axis=1, keepdims=True)
            for gk_ref in (gk0_ref, gk1_ref, gk2_ref,
                           gk3_ref, gk4_ref, gk5_ref)]
    ninf = jnp.full((Q, 1), -jnp.inf, jnp.float32)
    t1, t2, t3 = ninf, ninf, ninf
    for sv in sims:
        u = jnp.minimum(t1, sv)
        t1 = jnp.maximum(t1, sv)
        v = jnp.minimum(t2, u)
        t2 = jnp.maximum(t2, u)
        t3 = jnp.maximum(t3, v)
    e = [jnp.where(sv >= t3, jnp.exp(sv - t1), jnp.float32(0.0))
         for sv in sims]
    denom = e[0] + e[1] + e[2] + e[3] + e[4] + e[5]
    sim = (e[0] * gv_refs[0][...] + e[1] * gv_refs[1][...]
           + e[2] * gv_refs[2][...] + e[3] * gv_refs[3][...]
           + e[4] * gv_refs[4][...] + e[5] * gv_refs[5][...])
    sim = sim / denom
    base = posf_ref[...] * wpos_ref[...] + bpos_ref[...]
    fe = base + sw_ref[...] * sim
    gate_in = jnp.concatenate([fe, ts_ref[...]], axis=1)
    z = lax.dot_general(gate_in, wg_ref[...], (((1,), (1,)), ((), ())),
                        preferred_element_type=jnp.float32) + bg_ref[...]
    ti = jax.nn.sigmoid(z)
    ev = lax.dot_general(fe, we_ref[...], (((1,), (1,)), ((), ())),
                         preferred_element_type=jnp.float32) + be_ref[...]
    out_ref[...] = fe + ti * ev


def _epilogue_call(pos_f, wpos_row, bpos_row, ck, gvs, gks,
                   temporal_state, sw, W_gate, bg_row, W_evol, be_row):
    return pl.pallas_call(
        _epilogue_body,
        out_shape=jax.ShapeDtypeStruct((Q, D), jnp.float32),
    )(pos_f, wpos_row, bpos_row, ck, *gvs, *gks, temporal_state,
      sw, W_gate, bg_row, W_evol, be_row)


def kernel(positions, token_content, temporal_state, mem_keys, mem_values,
           W_pos, b_pos, W_content, b_content, similarity_weight,
           W_gate, b_gate, W_evol, b_evol):
    top_idx, ck = _topk_call(
        token_content, W_content, b_content.reshape(1, D), mem_keys)

    flat_idx = top_idx[:, :N_CAND].T.reshape(-1)
    gath_v, gath_k = _gather_call(flat_idx, mem_values, mem_keys)
    gvs = [gath_v[t * Q:(t + 1) * Q] for t in range(N_CAND)]
    gks = [gath_k[t * Q:(t + 1) * Q] for t in range(N_CAND)]

    pos_f = positions.astype(jnp.float32).reshape(Q, 1)
    return _epilogue_call(
        pos_f,
        W_pos.reshape(1, D),
        b_pos.reshape(1, D),
        ck,
        gvs,
        gks,
        temporal_state,
        similarity_weight.reshape(1, 1),
        W_gate,
        b_gate.reshape(1, D),
        W_evol,
        b_evol.reshape(1, D),
    )


# speedup vs baseline: 1.4122x; 1.0694x over previous
import functools

import jax
import jax.numpy as jnp
from jax import lax
from jax.experimental import pallas as pl
from jax.experimental.pallas import tpu as pltpu
from jax.experimental.pallas import tpu_sc as plsc

Q = 1024
K = 131072
D = 128
TOP_K = 3
BK = 4096
PAD = 8

N_CAND = 2 * TOP_K
_VMASK = -512
_SMASK = 511


def _topk_body(tc_ref, wc_ref, bc_ref, mk_ref, idx_ref, ck_ref, ckb_ref,
               m1_ref, m2_ref, m3_ref):
    k = pl.program_id(0)

    @pl.when(k == 0)
    def _init():
        ck = lax.dot_general(tc_ref[...], wc_ref[...], (((1,), (1,)), ((), ())),
                             preferred_element_type=jnp.float32)
        ck = ck + bc_ref[...]
        ck_ref[...] = ck
        ckb_ref[...] = ck.astype(jnp.bfloat16)
        m1_ref[...] = jnp.full((Q, 128), -jnp.inf, jnp.float32)
        m2_ref[...] = jnp.full((Q, 128), -jnp.inf, jnp.float32)
        m3_ref[...] = jnp.full((Q, 128), -jnp.inf, jnp.float32)

    mk = mk_ref[...].astype(jnp.bfloat16)
    s = lax.dot_general(ckb_ref[...], mk, (((1,), (1,)), ((), ())),
                        preferred_element_type=jnp.float32)

    def pair_key(j):
        p = jnp.maximum(s[:, (2 * j) * 128:(2 * j + 1) * 128],
                        s[:, (2 * j + 1) * 128:(2 * j + 2) * 128])
        xb = lax.bitcast_convert_type(p, jnp.int32)
        ki = (xb & jnp.int32(_VMASK)) | (k * (BK // 256) + j)
        return lax.bitcast_convert_type(ki, jnp.float32)

    m1, m2, m3 = m1_ref[...], m2_ref[...], m3_ref[...]
    for j in range(0, BK // 256, 2):
        x1 = pair_key(j)
        x2 = pair_key(j + 1)
        hi = jnp.maximum(x1, x2)
        lo = jnp.minimum(x1, x2)
        u = jnp.minimum(m1, hi)
        m1 = jnp.maximum(m1, hi)
        q = jnp.maximum(m2, lo)
        v = jnp.minimum(u, q)
        m2 = jnp.maximum(u, q)
        m3 = jnp.maximum(v, m3)
    m1_ref[...] = m1
    m2_ref[...] = m2
    m3_ref[...] = m3

    @pl.when(k == pl.num_programs(0) - 1)
    def _extract():
        iota = lax.broadcasted_iota(jnp.int32, (Q, 128), 1)
        cur, nxt, nxt2 = m1, m2, m3
        out = []
        for t in range(TOP_K):
            mx = jnp.max(cur, axis=1, keepdims=True)
            mi = lax.bitcast_convert_type(mx, jnp.int32)
            hit = cur == mx
            lane = jnp.min(jnp.where(hit, iota, jnp.int32(128)),
                           axis=1, keepdims=True)
            r0 = ((mi & jnp.int32(_SMASK)) << 8) | lane
            out.append(r0)
            out.append(r0 + jnp.int32(128))
            if t < TOP_K - 1:
                first = hit & (iota == lane)
                cur = jnp.where(first, nxt, cur)
                nxt = jnp.where(first, nxt2, nxt)
                nxt2 = jnp.where(first, jnp.float32(-jnp.inf), nxt2)
        pad_i = jnp.zeros((Q, PAD - N_CAND), jnp.int32)
        idx_ref[...] = jnp.concatenate(out + [pad_i], axis=1)


def _topk_call(token_content, W_content, b_content_row, mem_keys):
    return pl.pallas_call(
        _topk_body,
        grid=(K // BK,),
        in_specs=[
            pl.BlockSpec((Q, D), lambda k: (0, 0)),
            pl.BlockSpec((D, D), lambda k: (0, 0)),
            pl.BlockSpec((1, D), lambda k: (0, 0)),
            pl.BlockSpec((BK, D), lambda k: (k, 0)),
        ],
        out_specs=[
            pl.BlockSpec((Q, PAD), lambda k: (0, 0)),
            pl.BlockSpec((Q, D), lambda k: (0, 0)),
        ],
        out_shape=[
            jax.ShapeDtypeStruct((Q, PAD), jnp.int32),
            jax.ShapeDtypeStruct((Q, D), jnp.float32),
        ],
        scratch_shapes=[
            pltpu.VMEM((Q, D), jnp.bfloat16),
            pltpu.VMEM((Q, 128), jnp.float32),
            pltpu.VMEM((Q, 128), jnp.float32),
            pltpu.VMEM((Q, 128), jnp.float32),
        ],
        compiler_params=pltpu.CompilerParams(
            dimension_semantics=("arbitrary",)),
    )(token_content, W_content, b_content_row, mem_keys)


def _gather_call(flat_idx, values, keys):
    B = flat_idx.shape[0]
    info = plsc.get_sparse_core_info()
    nc, ns = info.num_cores, info.num_subcores
    nw = nc * ns
    b_per_w = B // nw
    mesh = plsc.VectorSubcoreMesh(core_axis_name="c", subcore_axis_name="s")

    @functools.partial(
        pl.kernel, mesh=mesh,
        out_type=[
            jax.ShapeDtypeStruct((B, D), jnp.float32),
            jax.ShapeDtypeStruct((B, D), jnp.float32),
        ],
        scratch_types=[
            pltpu.VMEM((b_per_w,), jnp.int32),
            pltpu.VMEM((b_per_w, D), jnp.float32),
            pltpu.VMEM((b_per_w, D), jnp.float32),
            pltpu.SemaphoreType.DMA,
            pltpu.SemaphoreType.DMA,
        ],
    )
    def gather_k(idx_hbm, val_hbm, key_hbm, outv_hbm, outk_hbm,
                 idx_v, rows_v, rows_k, semv, semk):
        wid = lax.axis_index("s") * nc + lax.axis_index("c")
        base = wid * b_per_w
        pltpu.sync_copy(idx_hbm.at[pl.ds(base, b_per_w)], idx_v)
        cpv = pltpu.async_copy(val_hbm.at[idx_v], rows_v, semv)
        cpk = pltpu.async_copy(key_hbm.at[idx_v], rows_k, semk)
        cpv.wait()
        cpk.wait()
        pltpu.sync_copy(rows_v, outv_hbm.at[pl.ds(base, b_per_w)])
        pltpu.sync_copy(rows_k, outk_hbm.at[pl.ds(base, b_per_w)])

    return gather_k(flat_idx, values, keys)


def _epilogue_body(posf_ref, wpos_ref, bpos_ref, ck_ref,
                   gv0_ref, gv1_ref, gv2_ref, gv3_ref, gv4_ref, gv5_ref,
                   gk0_ref, gk1_ref, gk2_ref, gk3_ref, gk4_ref, gk5_ref,
                   ts_ref, sw_ref, wg_ref, bg_ref, we_ref, be_ref, out_ref):
    ck = ck_ref[...]
    gv_refs = (gv0_ref, gv1_ref, gv2_ref, gv3_ref, gv4_ref, gv5_ref)
    sims = [jnp.sum(ck * gk_ref[...], axis=1, keepdims=True)
            for gk_ref in (gk0_ref, gk1_ref, gk2_ref,
                           gk3_ref, gk4_ref, gk5_ref)]
    ninf = jnp.full((Q, 1), -jnp.inf, jnp.float32)
    t1, t2, t3 = ninf, ninf, ninf
    for sv in sims:
        u = jnp.minimum(t1, sv)
        t1 = jnp.maximum(t1, sv)
        v = jnp.minimum(t2, u)
        t2 = jnp.maximum(t2, u)
        t3 = jnp.maximum(t3, v)
    e = [jnp.where(sv >= t3, jnp.exp(sv - t1), jnp.float32(0.0))
         for sv in sims]
    denom = e[0] + e[1] + e[2] + e[3] + e[4] + e[5]
    sim = (e[0] * gv_refs[0][...] + e[1] * gv_refs[1][...]
           + e[2] * gv_refs[2][...] + e[3] * gv_refs[3][...]
           + e[4] * gv_refs[4][...] + e[5] * gv_refs[5][...])
    sim = sim / denom
    base = posf_ref[...] * wpos_ref[...] + bpos_ref[...]
    fe = base + sw_ref[...] * sim
    gate_in = jnp.concatenate([fe, ts_ref[...]], axis=1)
    z = lax.dot_general(gate_in, wg_ref[...], (((1,), (1,)), ((), ())),
                        preferred_element_type=jnp.float32) + bg_ref[...]
    ti = jax.nn.sigmoid(z)
    ev = lax.dot_general(fe, we_ref[...], (((1,), (1,)), ((), ())),
                         preferred_element_type=jnp.float32) + be_ref[...]
    out_ref[...] = fe + ti * ev


def _epilogue_call(pos_f, wpos_row, bpos_row, ck, gvs, gks,
                   temporal_state, sw, W_gate, bg_row, W_evol, be_row):
    return pl.pallas_call(
        _epilogue_body,
        out_shape=jax.ShapeDtypeStruct((Q, D), jnp.float32),
    )(pos_f, wpos_row, bpos_row, ck, *gvs, *gks, temporal_state,
      sw, W_gate, bg_row, W_evol, be_row)


def kernel(positions, token_content, temporal_state, mem_keys, mem_values,
           W_pos, b_pos, W_content, b_content, similarity_weight,
           W_gate, b_gate, W_evol, b_evol):
    top_idx, ck = _topk_call(
        token_content, W_content, b_content.reshape(1, D), mem_keys)

    flat_idx = top_idx[:, :N_CAND].T.reshape(-1)
    gath_v, gath_k = _gather_call(flat_idx, mem_values, mem_keys)
    gvs = [gath_v[t * Q:(t + 1) * Q] for t in range(N_CAND)]
    gks = [gath_k[t * Q:(t + 1) * Q] for t in range(N_CAND)]

    pos_f = positions.astype(jnp.float32).reshape(Q, 1)
    return _epilogue_call(
        pos_f,
        W_pos.reshape(1, D),
        b_pos.reshape(1, D),
        ck,
        gvs,
        gks,
        temporal_state,
        similarity_weight.reshape(1, 1),
        W_gate,
        b_gate.reshape(1, D),
        W_evol,
        b_evol.reshape(1, D),
    )


# speedup vs baseline: 1.4765x; 1.0456x over previous
import functools

import jax
import jax.numpy as jnp
from jax import lax
from jax.experimental import pallas as pl
from jax.experimental.pallas import tpu as pltpu
from jax.experimental.pallas import tpu_sc as plsc

Q = 1024
K = 131072
D = 128
TOP_K = 3
BK = 8192
PAD = 8

N_CAND = 2 * TOP_K
_VMASK = -512
_SMASK = 511


def _topk_body(tc_ref, wc_ref, bc_ref, mk_ref, idx_ref, ck_ref, ckb_ref,
               m1_ref, m2_ref, m3_ref):
    k = pl.program_id(0)

    @pl.when(k == 0)
    def _init():
        ck = lax.dot_general(tc_ref[...], wc_ref[...], (((1,), (1,)), ((), ())),
                             preferred_element_type=jnp.float32)
        ck = ck + bc_ref[...]
        ck_ref[...] = ck
        ckb_ref[...] = ck.astype(jnp.bfloat16)
        m1_ref[...] = jnp.full((Q, 128), -jnp.inf, jnp.float32)
        m2_ref[...] = jnp.full((Q, 128), -jnp.inf, jnp.float32)
        m3_ref[...] = jnp.full((Q, 128), -jnp.inf, jnp.float32)

    mk = mk_ref[...].astype(jnp.bfloat16)
    s = lax.dot_general(ckb_ref[...], mk, (((1,), (1,)), ((), ())),
                        preferred_element_type=jnp.float32)

    def pair_key(j):
        p = jnp.maximum(s[:, (2 * j) * 128:(2 * j + 1) * 128],
                        s[:, (2 * j + 1) * 128:(2 * j + 2) * 128])
        xb = lax.bitcast_convert_type(p, jnp.int32)
        ki = (xb & jnp.int32(_VMASK)) | (k * (BK // 256) + j)
        return lax.bitcast_convert_type(ki, jnp.float32)

    m1, m2, m3 = m1_ref[...], m2_ref[...], m3_ref[...]
    for j in range(0, BK // 256, 2):
        x1 = pair_key(j)
        x2 = pair_key(j + 1)
        hi = jnp.maximum(x1, x2)
        lo = jnp.minimum(x1, x2)
        u = jnp.minimum(m1, hi)
        m1 = jnp.maximum(m1, hi)
        q = jnp.maximum(m2, lo)
        v = jnp.minimum(u, q)
        m2 = jnp.maximum(u, q)
        m3 = jnp.maximum(v, m3)
    m1_ref[...] = m1
    m2_ref[...] = m2
    m3_ref[...] = m3

    @pl.when(k == pl.num_programs(0) - 1)
    def _extract():
        iota = lax.broadcasted_iota(jnp.int32, (Q, 128), 1)
        cur, nxt, nxt2 = m1, m2, m3
        out = []
        for t in range(TOP_K):
            mx = jnp.max(cur, axis=1, keepdims=True)
            mi = lax.bitcast_convert_type(mx, jnp.int32)
            hit = cur == mx
            lane = jnp.min(jnp.where(hit, iota, jnp.int32(128)),
                           axis=1, keepdims=True)
            r0 = ((mi & jnp.int32(_SMASK)) << 8) | lane
            out.append(r0)
            out.append(r0 + jnp.int32(128))
            if t < TOP_K - 1:
                first = hit & (iota == lane)
                cur = jnp.where(first, nxt, cur)
                nxt = jnp.where(first, nxt2, nxt)
                nxt2 = jnp.where(first, jnp.float32(-jnp.inf), nxt2)
        pad_i = jnp.zeros((Q, PAD - N_CAND), jnp.int32)
        idx_ref[...] = jnp.concatenate(out + [pad_i], axis=1)


def _topk_call(token_content, W_content, b_content_row, mem_keys):
    return pl.pallas_call(
        _topk_body,
        grid=(K // BK,),
        in_specs=[
            pl.BlockSpec((Q, D), lambda k: (0, 0)),
            pl.BlockSpec((D, D), lambda k: (0, 0)),
            pl.BlockSpec((1, D), lambda k: (0, 0)),
            pl.BlockSpec((BK, D), lambda k: (k, 0)),
        ],
        out_specs=[
            pl.BlockSpec((Q, PAD), lambda k: (0, 0)),
            pl.BlockSpec((Q, D), lambda k: (0, 0)),
        ],
        out_shape=[
            jax.ShapeDtypeStruct((Q, PAD), jnp.int32),
            jax.ShapeDtypeStruct((Q, D), jnp.float32),
        ],
        scratch_shapes=[
            pltpu.VMEM((Q, D), jnp.bfloat16),
            pltpu.VMEM((Q, 128), jnp.float32),
            pltpu.VMEM((Q, 128), jnp.float32),
            pltpu.VMEM((Q, 128), jnp.float32),
        ],
        compiler_params=pltpu.CompilerParams(
            dimension_semantics=("arbitrary",)),
    )(token_content, W_content, b_content_row, mem_keys)


def _gather_call(flat_idx, values, keys):
    B = flat_idx.shape[0]
    info = plsc.get_sparse_core_info()
    nc, ns = info.num_cores, info.num_subcores
    nw = nc * ns
    b_per_w = B // nw
    mesh = plsc.VectorSubcoreMesh(core_axis_name="c", subcore_axis_name="s")

    @functools.partial(
        pl.kernel, mesh=mesh,
        out_type=[
            jax.ShapeDtypeStruct((B, D), jnp.float32),
            jax.ShapeDtypeStruct((B, D), jnp.float32),
        ],
        scratch_types=[
            pltpu.VMEM((b_per_w,), jnp.int32),
            pltpu.VMEM((b_per_w, D), jnp.float32),
            pltpu.VMEM((b_per_w, D), jnp.float32),
            pltpu.SemaphoreType.DMA,
            pltpu.SemaphoreType.DMA,
        ],
    )
    def gather_k(idx_hbm, val_hbm, key_hbm, outv_hbm, outk_hbm,
                 idx_v, rows_v, rows_k, semv, semk):
        wid = lax.axis_index("s") * nc + lax.axis_index("c")
        base = wid * b_per_w
        pltpu.sync_copy(idx_hbm.at[pl.ds(base, b_per_w)], idx_v)
        cpv = pltpu.async_copy(val_hbm.at[idx_v], rows_v, semv)
        cpk = pltpu.async_copy(key_hbm.at[idx_v], rows_k, semk)
        cpv.wait()
        cpk.wait()
        pltpu.sync_copy(rows_v, outv_hbm.at[pl.ds(base, b_per_w)])
        pltpu.sync_copy(rows_k, outk_hbm.at[pl.ds(base, b_per_w)])

    return gather_k(flat_idx, values, keys)


def _epilogue_body(posf_ref, wpos_ref, bpos_ref, ck_ref,
                   gv0_ref, gv1_ref, gv2_ref, gv3_ref, gv4_ref, gv5_ref,
                   gk0_ref, gk1_ref, gk2_ref, gk3_ref, gk4_ref, gk5_ref,
                   ts_ref, sw_ref, wg_ref, bg_ref, we_ref, be_ref, out_ref):
    ck = ck_ref[...]
    gv_refs = (gv0_ref, gv1_ref, gv2_ref, gv3_ref, gv4_ref, gv5_ref)
    sims = [jnp.sum(ck * gk_ref[...], axis=1, keepdims=True)
            for gk_ref in (gk0_ref, gk1_ref, gk2_ref,
                           gk3_ref, gk4_ref, gk5_ref)]
    ninf = jnp.full((Q, 1), -jnp.inf, jnp.float32)
    t1, t2, t3 = ninf, ninf, ninf
    for sv in sims:
        u = jnp.minimum(t1, sv)
        t1 = jnp.maximum(t1, sv)
        v = jnp.minimum(t2, u)
        t2 = jnp.maximum(t2, u)
        t3 = jnp.maximum(t3, v)
    e = [jnp.where(sv >= t3, jnp.exp(sv - t1), jnp.float32(0.0))
         for sv in sims]
    denom = e[0] + e[1] + e[2] + e[3] + e[4] + e[5]
    sim = (e[0] * gv_refs[0][...] + e[1] * gv_refs[1][...]
           + e[2] * gv_refs[2][...] + e[3] * gv_refs[3][...]
           + e[4] * gv_refs[4][...] + e[5] * gv_refs[5][...])
    sim = sim / denom
    base = posf_ref[...] * wpos_ref[...] + bpos_ref[...]
    fe = base + sw_ref[...] * sim
    gate_in = jnp.concatenate([fe, ts_ref[...]], axis=1)
    z = lax.dot_general(gate_in, wg_ref[...], (((1,), (1,)), ((), ())),
                        preferred_element_type=jnp.float32) + bg_ref[...]
    ti = jax.nn.sigmoid(z)
    ev = lax.dot_general(fe, we_ref[...], (((1,), (1,)), ((), ())),
                         preferred_element_type=jnp.float32) + be_ref[...]
    out_ref[...] = fe + ti * ev


def _epilogue_call(pos_f, wpos_row, bpos_row, ck, gvs, gks,
                   temporal_state, sw, W_gate, bg_row, W_evol, be_row):
    return pl.pallas_call(
        _epilogue_body,
        out_shape=jax.ShapeDtypeStruct((Q, D), jnp.float32),
    )(pos_f, wpos_row, bpos_row, ck, *gvs, *gks, temporal_state,
      sw, W_gate, bg_row, W_evol, be_row)


def kernel(positions, token_content, temporal_state, mem_keys, mem_values,
           W_pos, b_pos, W_content, b_content, similarity_weight,
           W_gate, b_gate, W_evol, b_evol):
    top_idx, ck = _topk_call(
        token_content, W_content, b_content.reshape(1, D), mem_keys)

    flat_idx = top_idx[:, :N_CAND].T.reshape(-1)
    gath_v, gath_k = _gather_call(flat_idx, mem_values, mem_keys)
    gvs = [gath_v[t * Q:(t + 1) * Q] for t in range(N_CAND)]
    gks = [gath_k[t * Q:(t + 1) * Q] for t in range(N_CAND)]

    pos_f = positions.astype(jnp.float32).reshape(Q, 1)
    return _epilogue_call(
        pos_f,
        W_pos.reshape(1, D),
        b_pos.reshape(1, D),
        ck,
        gvs,
        gks,
        temporal_state,
        similarity_weight.reshape(1, 1),
        W_gate,
        b_gate.reshape(1, D),
        W_evol,
        b_evol.reshape(1, D),
    )
